# baseline (speedup 1.0000x reference)
import jax
import jax.numpy as jnp
from jax.experimental import pallas as pl
from jax.experimental.pallas import tpu as pltpu

_CONTRACT0 = (((0,), (0,)), ((), ()))
_BLK = 128


def _net_kernel(data_ref, matrix_ref, conv_W_ref, conv_b_ref,
                fc1_W_ref, fc1_b_ref, fc2_W_ref, fc2_b_ref, out_ref,
                a_bf_ref, colsum_ref, xw_ref):
    f32, bf16 = jnp.float32, jnp.bfloat16
    k = pl.program_id(0)
    nsteps = pl.num_programs(0)

    blk = matrix_ref[...].astype(f32)
    part = jnp.sum(blk, axis=0, keepdims=True)

    @pl.when(k == 0)
    def _init():
        colsum_ref[...] = part
        xw_ref[...] = jnp.dot(data_ref[...], conv_W_ref[...],
                              preferred_element_type=f32)

    @pl.when(k > 0)
    def _acc():
        colsum_ref[...] += part

    a_bf_ref[pl.ds(k * _BLK, _BLK), :] = blk.astype(bf16)

    @pl.when(k == nsteps - 1)
    def _finish():
        deg = colsum_ref[...] + 1.0
        dinv = jnp.transpose(jax.lax.rsqrt(deg))
        z = xw_ref[...] * dinv
        z_hi = z.astype(bf16)
        z_lo = (z - z_hi.astype(f32)).astype(bf16)
        rhs = jnp.concatenate([z_hi, z_lo], axis=1)
        agg2 = jax.lax.dot_general(a_bf_ref[...], rhs, _CONTRACT0,
                                   preferred_element_type=f32)
        h = agg2[:, :z.shape[1]] + agg2[:, z.shape[1]:] + z
        h = jnp.maximum(h * dinv + conv_b_ref[...], 0.0)
        h = jnp.maximum(jnp.dot(h, fc1_W_ref[...],
                                preferred_element_type=f32) + fc1_b_ref[...], 0.0)
        out_ref[...] = jnp.dot(h, fc2_W_ref[...],
                               preferred_element_type=f32) + fc2_b_ref[...]


def kernel(data, matrix, conv_W, conv_b, fc1_W, fc1_b, fc2_W, fc2_b):
    n, d = data.shape
    h = conv_W.shape[1]
    o = fc2_W.shape[1]
    nsteps = n // _BLK
    fixed = lambda i, j: (lambda k: (i, j))
    return pl.pallas_call(
        _net_kernel,
        grid=(nsteps,),
        in_specs=[
            pl.BlockSpec((n, d), fixed(0, 0)),
            pl.BlockSpec((_BLK, n), lambda k: (k, 0)),
            pl.BlockSpec((d, h), fixed(0, 0)),
            pl.BlockSpec((1, h), fixed(0, 0)),
            pl.BlockSpec((h, h), fixed(0, 0)),
            pl.BlockSpec((1, h), fixed(0, 0)),
            pl.BlockSpec((h, o), fixed(0, 0)),
            pl.BlockSpec((1, o), fixed(0, 0)),
        ],
        out_specs=pl.BlockSpec((n, o), fixed(0, 0)),
        out_shape=jax.ShapeDtypeStruct((n, o), jnp.float32),
        scratch_shapes=[
            pltpu.VMEM((n, n), jnp.bfloat16),
            pltpu.VMEM((1, n), jnp.float32),
            pltpu.VMEM((n, h), jnp.float32),
        ],
    )(data, matrix, conv_W, conv_b.reshape(1, -1),
      fc1_W, fc1_b.reshape(1, -1), fc2_W, fc2_b.reshape(1, -1))
